# baseline (speedup 1.0000x reference)
import jax
import jax.numpy as jnp
from jax import lax
from jax.experimental import pallas as pl
from jax.experimental.pallas import tpu as pltpu
from jax.experimental.pallas import tpu_sc as plsc

VOCAB = 1000000
DIM = 64
B = 1024
L = 200
NW = 32
SEQW = B // NW
TOKW = SEQW * L
CH = 128
NCH = TOKW // CH
NIF = 5
NSLOT = 2 * NIF
NOUT = NCH // NSLOT

SEQB = 32
TCB = SEQB * L

_EPS = 1e-5
_SCALE = 8.0


def _pos_encoding():
    dim_loc = jnp.arange(0, DIM, 2).astype(jnp.float32)
    pos_loc = jnp.arange(0, L).astype(jnp.float32)
    denominator = jnp.exp(-(dim_loc / DIM) * jnp.log(10000.0))
    ang = pos_loc[:, None] * denominator[None, :]
    pe = jnp.zeros((L, DIM), dtype=jnp.float32)
    pe = pe.at[:, 0::2].set(jnp.sin(ang))
    pe = pe.at[:, 1::2].set(jnp.cos(ang))
    return pe


def _worker_id():
    return lax.axis_index("s") * 2 + lax.axis_index("c")


def _sc_gather(x_hbm, table_hbm, out_hbm, idx2d, rows, gsem, osem):
    wid = _worker_id()
    pltpu.sync_copy(x_hbm.at[wid], idx2d)

    for b in range(NIF):
        pltpu.make_async_copy(
            table_hbm.at[idx2d.at[b]], rows[b], gsem[b]).start()

    def outer(c0, _):
        for b in range(NSLOT):
            c = c0 * NSLOT + b
            pltpu.make_async_copy(
                table_hbm.at[idx2d.at[c]], rows[b], gsem[b]).wait()
            pltpu.make_async_copy(
                rows[b], out_hbm.at[wid, c], osem[b]).start()

            s2 = (b + NIF) % NSLOT

            @pl.when(c + NIF < NCH)
            def _issue():
                @pl.when(c >= NIF)
                def _drain():
                    pltpu.make_async_copy(
                        rows[s2], out_hbm.at[wid, c - NIF], osem[s2]).wait()
                pltpu.make_async_copy(
                    table_hbm.at[idx2d.at[c + NIF]], rows[s2],
                    gsem[s2]).start()
        return 0

    lax.fori_loop(0, NOUT, outer, 0)

    for b in range(NSLOT):
        c = (NOUT - 1) * NSLOT + b
        pltpu.make_async_copy(
            rows[b], out_hbm.at[wid, c], osem[b]).wait()


def _tc_norm(emb_ref, pe_ref, g_ref, b_ref, out_ref):
    e = emb_ref[...] * _SCALE + pe_ref[...]
    mean = jnp.mean(e, axis=-1, keepdims=True)
    var = jnp.mean(e * e, axis=-1, keepdims=True) - mean * mean
    out_ref[...] = (e - mean) * lax.rsqrt(var + _EPS) * g_ref[...] + b_ref[...]


def kernel(x, table, ln_gamma, ln_beta):
    xw = x.astype(jnp.int32).reshape(NW, NCH, CH)

    mesh = plsc.VectorSubcoreMesh(core_axis_name="c", subcore_axis_name="s")
    gather = pl.kernel(
        _sc_gather,
        out_type=jax.ShapeDtypeStruct((NW, NCH, CH, DIM), jnp.float32),
        mesh=mesh,
        compiler_params=pltpu.CompilerParams(
            needs_layout_passes=False, use_tc_tiling_on_sc=False),
        scratch_types=[
            pltpu.VMEM((NCH, CH), jnp.int32),
            [pltpu.VMEM((CH, DIM), jnp.float32) for _ in range(NSLOT)],
            [pltpu.SemaphoreType.DMA for _ in range(NSLOT)],
            [pltpu.SemaphoreType.DMA for _ in range(NSLOT)],
        ],
    )
    emb = gather(xw, table).reshape(B * L, DIM)

    pe_rep = jnp.tile(_pos_encoding(), (SEQB, 1))
    norm = pl.pallas_call(
        _tc_norm,
        grid=(B // SEQB,),
        in_specs=[
            pl.BlockSpec((TCB, DIM), lambda i: (i, 0)),
            pl.BlockSpec((TCB, DIM), lambda i: (0, 0)),
            pl.BlockSpec((1, DIM), lambda i: (0, 0)),
            pl.BlockSpec((1, DIM), lambda i: (0, 0)),
        ],
        out_specs=pl.BlockSpec((TCB, DIM), lambda i: (i, 0)),
        out_shape=jax.ShapeDtypeStruct((B * L, DIM), jnp.float32),
    )(emb, pe_rep, ln_gamma.reshape(1, DIM), ln_beta.reshape(1, DIM))
    return norm.reshape(B, L, DIM)
